# Initial kernel scaffold; baseline (speedup 1.0000x reference)
import jax
import jax.numpy as jnp
from jax.experimental import pallas as pl


def kernel(users, edge_index, edge_weight, user_emb, item_emb):
    raise NotImplementedError("write your pallas kernel here")


# speedup vs baseline: 2.9550x; 2.9550x over previous
import jax
import jax.numpy as jnp
from jax import lax
from jax.experimental import pallas as pl
from jax.experimental.pallas import tpu as pltpu
from jax.experimental.pallas import tpu_sc as plsc

NUM_USERS = 25000
NUM_ITEMS = 25000
N_TOTAL = NUM_USERS + NUM_ITEMS
D = 64
N_LAYERS = 3
N_EDGES = 800000
BATCH = 1024

NC = 2
NS = 16
NW = NC * NS
LANES = 16

HALF = NUM_USERS
TRASH = HALF
ACC_ROWS = HALF + 8

CHUNK = 128
EDGES_PER_TILE = -(-N_EDGES // NS)
CHUNKS_PER_TILE = -(-EDGES_PER_TILE // CHUNK)
EPT_PAD = CHUNKS_PER_TILE * CHUNK
E_PAD = EPT_PAD * NS

ACC_BLK = 128
N_ACC_BLOCKS = HALF // ACC_BLK
ACC_REM = HALF - N_ACC_BLOCKS * ACC_BLK
K_BLOCKS = (N_ACC_BLOCKS + NS - 1) // NS

USERS_PER_TILE = BATCH // NW

_MESH = dict(core_axis_name="c", subcore_axis_name="s")
_CP = pltpu.CompilerParams(use_tc_tiling_on_sc=False)


def _prop_body(table_hbm, esrc_hbm, edst_hbm, ew_hbm, zblk_hbm,
               out_hbm,
               sidx_v, sdst_v, w_v, rows_v, zblk_v, acc_sh, sem):
    c = lax.axis_index("c")
    s = lax.axis_index("s")
    lo = c * HALF
    base = s * EPT_PAD

    pltpu.sync_copy(zblk_hbm, zblk_v)
    for k in range(K_BLOCKS):
        b = s + k * NS

        @pl.when(b < N_ACC_BLOCKS)
        def _():
            pltpu.sync_copy(zblk_v, acc_sh.at[pl.ds(b * ACC_BLK, ACC_BLK)])

    @pl.when(s == NS - 1)
    def _():
        pltpu.sync_copy(
            zblk_v.at[pl.ds(0, ACC_REM + (ACC_ROWS - HALF))],
            acc_sh.at[pl.ds(N_ACC_BLOCKS * ACC_BLK,
                            ACC_REM + (ACC_ROWS - HALF))])

    plsc.subcore_barrier()

    def chunk_body(ch, carry):
        e0 = base + ch * CHUNK
        pltpu.sync_copy(esrc_hbm.at[pl.ds(e0, CHUNK)], sidx_v)
        pltpu.sync_copy(edst_hbm.at[pl.ds(e0, CHUNK)], sdst_v.at[0])
        pltpu.sync_copy(ew_hbm.at[pl.ds(e0, CHUNK)], w_v)
        pltpu.async_copy(table_hbm.at[sidx_v], rows_v, sem).wait()
        for g in range(CHUNK // LANES):
            sl = pl.ds(g * LANES, LANES)
            dl = sdst_v[0, sl] - lo
            m = (dl >= 0) & (dl < HALF)
            sdst_v[0, sl] = jnp.where(m, dl, TRASH)
            wm = jnp.where(m, w_v[sl], 0.0)
            w_v[sl] = wm
            for e in range(LANES):
                wb = jnp.broadcast_to(wm[e], (LANES,))
                row = g * LANES + e
                for q in range(D // LANES):
                    qsl = pl.ds(q * LANES, LANES)
                    rows_v[row, qsl] = rows_v[row, qsl] * wb
        pltpu.sync_copy(rows_v, acc_sh.at[sdst_v.at[0]], add=True)
        return carry

    lax.fori_loop(0, CHUNKS_PER_TILE, chunk_body, jnp.int32(0))

    plsc.subcore_barrier()

    row0 = c * HALF
    for k in range(K_BLOCKS):
        b = s + k * NS

        @pl.when(b < N_ACC_BLOCKS)
        def _():
            pltpu.sync_copy(acc_sh.at[pl.ds(b * ACC_BLK, ACC_BLK)],
                            out_hbm.at[pl.ds(row0 + b * ACC_BLK, ACC_BLK)])

    @pl.when(s == NS - 1)
    def _():
        pltpu.sync_copy(
            acc_sh.at[pl.ds(N_ACC_BLOCKS * ACC_BLK, ACC_REM)],
            out_hbm.at[pl.ds(row0 + N_ACC_BLOCKS * ACC_BLK, ACC_REM)])


_prop = pl.kernel(
    _prop_body,
    out_type=jax.ShapeDtypeStruct((N_TOTAL, D), jnp.float32),
    mesh=plsc.VectorSubcoreMesh(**_MESH),
    compiler_params=_CP,
    scratch_types=[
        pltpu.VMEM((CHUNK,), jnp.int32),
        pltpu.VMEM((1, CHUNK), jnp.int32),
        pltpu.VMEM((CHUNK,), jnp.float32),
        pltpu.VMEM((CHUNK, D), jnp.float32),
        pltpu.VMEM((ACC_BLK, D), jnp.float32),
        pltpu.VMEM_SHARED((ACC_ROWS, D), jnp.float32),
        pltpu.SemaphoreType.DMA,
    ],
)


def _users_body(users_hbm, t0_hbm, t1_hbm, t2_hbm, t3_hbm, out_hbm,
                idx_v, acc_v, tmp_v, sem):
    c = lax.axis_index("c")
    s = lax.axis_index("s")
    wid = s * NC + c
    base = wid * USERS_PER_TILE

    pltpu.sync_copy(users_hbm.at[pl.ds(base, USERS_PER_TILE)], idx_v)
    pltpu.async_copy(t0_hbm.at[idx_v], acc_v, sem).wait()
    for t_hbm in (t1_hbm, t2_hbm, t3_hbm):
        pltpu.async_copy(t_hbm.at[idx_v], tmp_v, sem).wait()
        for r in range(USERS_PER_TILE):
            for q in range(D // LANES):
                sl = pl.ds(q * LANES, LANES)
                acc_v[r, sl] = acc_v[r, sl] + tmp_v[r, sl]
    for r in range(USERS_PER_TILE):
        for q in range(D // LANES):
            sl = pl.ds(q * LANES, LANES)
            acc_v[r, sl] = acc_v[r, sl] * 0.25
    pltpu.sync_copy(acc_v, out_hbm.at[pl.ds(base, USERS_PER_TILE)])


_users_gather = pl.kernel(
    _users_body,
    out_type=jax.ShapeDtypeStruct((BATCH, D), jnp.float32),
    mesh=plsc.VectorSubcoreMesh(**_MESH),
    compiler_params=_CP,
    scratch_types=[
        pltpu.VMEM((USERS_PER_TILE,), jnp.int32),
        pltpu.VMEM((USERS_PER_TILE, D), jnp.float32),
        pltpu.VMEM((USERS_PER_TILE, D), jnp.float32),
        pltpu.SemaphoreType.DMA,
    ],
)


BLK_I = 512
N_I_BLOCKS = -(-NUM_ITEMS // BLK_I)


def _rating_body(u_ref, b0, b1, b2, b3, o_ref):
    it = (b0[...] + b1[...] + b2[...] + b3[...]) * 0.25
    z = lax.dot_general(u_ref[...], it, (((1,), (1,)), ((), ())),
                        preferred_element_type=jnp.float32)
    o_ref[...] = 1.0 / (1.0 + jnp.exp(-z))


def _rating(users_emb, i0, i1, i2, i3):
    tspec = pl.BlockSpec((BLK_I, D), lambda j: (j, 0))
    return pl.pallas_call(
        _rating_body,
        grid=(N_I_BLOCKS,),
        in_specs=[
            pl.BlockSpec((BATCH, D), lambda j: (0, 0)),
            tspec, tspec, tspec, tspec,
        ],
        out_specs=pl.BlockSpec((BATCH, BLK_I), lambda j: (0, j)),
        out_shape=jax.ShapeDtypeStruct((BATCH, NUM_ITEMS), jnp.float32),
    )(users_emb, i0, i1, i2, i3)


def kernel(users, edge_index, edge_weight, user_emb, item_emb):
    t0 = jnp.concatenate([user_emb, item_emb], axis=0)
    pad = E_PAD - N_EDGES
    esrc = jnp.pad(edge_index[0], (0, pad))
    edst = jnp.pad(edge_index[1], (0, pad))
    ew = jnp.pad(edge_weight, (0, pad))
    zblk = jnp.zeros((ACC_BLK, D), jnp.float32)

    tables = [t0]
    t = t0
    for _ in range(N_LAYERS):
        t = _prop(t, esrc, edst, ew, zblk)
        tables.append(t)

    users_emb = _users_gather(users, *tables)
    items = [t[NUM_USERS:] for t in tables]
    return _rating(users_emb, *items)


# speedup vs baseline: 4.9984x; 1.6915x over previous
import jax
import jax.numpy as jnp
from jax import lax
from jax.experimental import pallas as pl
from jax.experimental.pallas import tpu as pltpu
from jax.experimental.pallas import tpu_sc as plsc

NUM_USERS = 25000
NUM_ITEMS = 25000
N_TOTAL = NUM_USERS + NUM_ITEMS
D = 64
N_LAYERS = 3
N_EDGES = 800000
BATCH = 1024

NC = 2
NS = 16
NW = NC * NS
LANES = 16

HALF = NUM_USERS
TRASH = HALF
ACC_ROWS = HALF + 8

CHUNK = 128
NBUF = 2
EDGES_PER_TILE = -(-N_EDGES // NS)
CHT = -(-EDGES_PER_TILE // CHUNK)
CHT = -(-CHT // NBUF) * NBUF
NBODIES = CHT // NBUF
EPT_PAD = CHT * CHUNK
E_PAD = EPT_PAD * NS + NBUF * CHUNK

ACC_BLK = 128
N_ACC_BLOCKS = HALF // ACC_BLK
ACC_REM = HALF - N_ACC_BLOCKS * ACC_BLK
K_BLOCKS = (N_ACC_BLOCKS + NS - 1) // NS

USERS_PER_TILE = BATCH // NW

_MESH = dict(core_axis_name="c", subcore_axis_name="s")
_CP = pltpu.CompilerParams(use_tc_tiling_on_sc=False)


def _prop_body(table_hbm, esrc_hbm, edst_hbm, ew_hbm, zblk_hbm,
               out_hbm,
               sidx8, sdst8, w8, didx8, rows8, acc_sh,
               tsem, gsem, ssem):
    c = lax.axis_index("c")
    s = lax.axis_index("s")
    lo = c * HALF
    base = s * EPT_PAD

    for k in range(K_BLOCKS):
        b = s + k * NS

        @pl.when(b < N_ACC_BLOCKS)
        def _():
            pltpu.sync_copy(zblk_hbm, acc_sh.at[pl.ds(b * ACC_BLK, ACC_BLK)])

    @pl.when(s == NS - 1)
    def _():
        pltpu.sync_copy(
            zblk_hbm.at[pl.ds(0, ACC_REM + (ACC_ROWS - HALF))],
            acc_sh.at[pl.ds(N_ACC_BLOCKS * ACC_BLK,
                            ACC_REM + (ACC_ROWS - HALF))])

    plsc.subcore_barrier()

    def fire_stage(i, b):
        e0 = base + i * CHUNK
        pltpu.async_copy(esrc_hbm.at[pl.ds(e0, CHUNK)], sidx8.at[b],
                         tsem.at[b])
        pltpu.async_copy(edst_hbm.at[pl.ds(e0, CHUNK)], sdst8.at[b],
                         tsem.at[b])
        pltpu.async_copy(ew_hbm.at[pl.ds(e0, CHUNK)], w8.at[b], tsem.at[b])

    def drain_stage(b):
        z = pl.ds(0, CHUNK)
        pltpu.make_async_copy(esrc_hbm.at[z], sidx8.at[b], tsem.at[b]).wait()
        pltpu.make_async_copy(edst_hbm.at[z], sdst8.at[b], tsem.at[b]).wait()
        pltpu.make_async_copy(ew_hbm.at[z], w8.at[b], tsem.at[b]).wait()

    def fire_gather(b):
        pltpu.async_copy(table_hbm.at[sidx8.at[b]], rows8.at[b], gsem.at[b])

    def drain_gather(b):
        pltpu.make_async_copy(table_hbm.at[sidx8.at[b]], rows8.at[b],
                              gsem.at[b]).wait()

    def fire_scatter(b):
        pltpu.async_copy(rows8.at[b], acc_sh.at[didx8.at[b]], ssem.at[b],
                         add=True)

    def drain_scatter(b):
        pltpu.make_async_copy(rows8.at[b], acc_sh.at[didx8.at[b]],
                              ssem.at[b]).wait()

    for b in range(NBUF):
        fire_stage(b, b)
    for b in range(NBUF):
        drain_stage(b)
        fire_gather(b)

    def body(p, carry):
        g0 = p * NBUF
        for b in range(NBUF):
            drain_gather(b)
            for g in range(CHUNK // LANES):
                sl = pl.ds(g * LANES, LANES)
                dl = sdst8[b, sl] - lo
                m = (dl >= 0) & (dl < HALF)
                didx8[b, sl] = jnp.where(m, dl, TRASH)
                wm = jnp.where(m, w8[b, sl], 0.0)
                for e in range(LANES):
                    wb = jnp.broadcast_to(wm[e], (LANES,))
                    row = g * LANES + e
                    for q in range(D // LANES):
                        qsl = pl.ds(q * LANES, LANES)
                        rows8[b, row, qsl] = rows8[b, row, qsl] * wb
            fire_scatter(b)
            fire_stage(g0 + b + NBUF, b)
        for b in range(NBUF):
            drain_stage(b)
            drain_scatter(b)
            fire_gather(b)
        return carry

    lax.fori_loop(0, NBODIES, body, jnp.int32(0))

    for b in range(NBUF):
        drain_gather(b)

    plsc.subcore_barrier()

    row0 = c * HALF
    for k in range(K_BLOCKS):
        b = s + k * NS

        @pl.when(b < N_ACC_BLOCKS)
        def _():
            pltpu.sync_copy(acc_sh.at[pl.ds(b * ACC_BLK, ACC_BLK)],
                            out_hbm.at[pl.ds(row0 + b * ACC_BLK, ACC_BLK)])

    @pl.when(s == NS - 1)
    def _():
        pltpu.sync_copy(
            acc_sh.at[pl.ds(N_ACC_BLOCKS * ACC_BLK, ACC_REM)],
            out_hbm.at[pl.ds(row0 + N_ACC_BLOCKS * ACC_BLK, ACC_REM)])


_prop = pl.kernel(
    _prop_body,
    out_type=jax.ShapeDtypeStruct((N_TOTAL, D), jnp.float32),
    mesh=plsc.VectorSubcoreMesh(**_MESH),
    compiler_params=_CP,
    scratch_types=[
        pltpu.VMEM((NBUF, CHUNK), jnp.int32),
        pltpu.VMEM((NBUF, CHUNK), jnp.int32),
        pltpu.VMEM((NBUF, CHUNK), jnp.float32),
        pltpu.VMEM((NBUF, CHUNK), jnp.int32),
        pltpu.VMEM((NBUF, CHUNK, D), jnp.float32),
        pltpu.VMEM_SHARED((ACC_ROWS, D), jnp.float32),
        pltpu.SemaphoreType.DMA((NBUF,)),
        pltpu.SemaphoreType.DMA((NBUF,)),
        pltpu.SemaphoreType.DMA((NBUF,)),
    ],
)


def _users_body(users_hbm, t0_hbm, t1_hbm, t2_hbm, t3_hbm, out_hbm,
                idx_v, acc_v, tmp_v, sem):
    c = lax.axis_index("c")
    s = lax.axis_index("s")
    wid = s * NC + c
    base = wid * USERS_PER_TILE

    pltpu.sync_copy(users_hbm.at[pl.ds(base, USERS_PER_TILE)], idx_v)
    pltpu.async_copy(t0_hbm.at[idx_v], acc_v, sem).wait()
    for t_hbm in (t1_hbm, t2_hbm, t3_hbm):
        pltpu.async_copy(t_hbm.at[idx_v], tmp_v, sem).wait()
        for r in range(USERS_PER_TILE):
            for q in range(D // LANES):
                sl = pl.ds(q * LANES, LANES)
                acc_v[r, sl] = acc_v[r, sl] + tmp_v[r, sl]
    for r in range(USERS_PER_TILE):
        for q in range(D // LANES):
            sl = pl.ds(q * LANES, LANES)
            acc_v[r, sl] = acc_v[r, sl] * 0.25
    pltpu.sync_copy(acc_v, out_hbm.at[pl.ds(base, USERS_PER_TILE)])


_users_gather = pl.kernel(
    _users_body,
    out_type=jax.ShapeDtypeStruct((BATCH, D), jnp.float32),
    mesh=plsc.VectorSubcoreMesh(**_MESH),
    compiler_params=_CP,
    scratch_types=[
        pltpu.VMEM((USERS_PER_TILE,), jnp.int32),
        pltpu.VMEM((USERS_PER_TILE, D), jnp.float32),
        pltpu.VMEM((USERS_PER_TILE, D), jnp.float32),
        pltpu.SemaphoreType.DMA,
    ],
)


BLK_I = 512
N_I_BLOCKS = -(-NUM_ITEMS // BLK_I)


def _rating_body(u_ref, b0, b1, b2, b3, o_ref):
    it = (b0[...] + b1[...] + b2[...] + b3[...]) * 0.25
    z = lax.dot_general(u_ref[...], it, (((1,), (1,)), ((), ())),
                        preferred_element_type=jnp.float32)
    o_ref[...] = 1.0 / (1.0 + jnp.exp(-z))


def _rating(users_emb, i0, i1, i2, i3):
    tspec = pl.BlockSpec((BLK_I, D), lambda j: (j, 0))
    return pl.pallas_call(
        _rating_body,
        grid=(N_I_BLOCKS,),
        in_specs=[
            pl.BlockSpec((BATCH, D), lambda j: (0, 0)),
            tspec, tspec, tspec, tspec,
        ],
        out_specs=pl.BlockSpec((BATCH, BLK_I), lambda j: (0, j)),
        out_shape=jax.ShapeDtypeStruct((BATCH, NUM_ITEMS), jnp.float32),
    )(users_emb, i0, i1, i2, i3)


def kernel(users, edge_index, edge_weight, user_emb, item_emb):
    t0 = jnp.concatenate([user_emb, item_emb], axis=0)
    pad = E_PAD - N_EDGES
    esrc = jnp.pad(edge_index[0], (0, pad))
    edst = jnp.pad(edge_index[1], (0, pad))
    ew = jnp.pad(edge_weight, (0, pad))
    zblk = jnp.zeros((ACC_BLK, D), jnp.float32)

    tables = [t0]
    t = t0
    for _ in range(N_LAYERS):
        t = _prop(t, esrc, edst, ew, zblk)
        tables.append(t)

    users_emb = _users_gather(users, *tables)
    items = [t[NUM_USERS:] for t in tables]
    return _rating(users_emb, *items)
